# Initial kernel scaffold; baseline (speedup 1.0000x reference)
import functools

import jax
import jax.numpy as jnp
from jax import lax
from jax.experimental import pallas as pl
from jax.experimental.pallas import tpu as pltpu
from jax.experimental.pallas import tpu_sc as plsc

_T, _H, _W = 256, 128, 128
_HW = _H * _W
_FRAMES_PER_WORKER = 16
_NUM_WORKERS = _T // _FRAMES_PER_WORKER


def _argmax_body(hm_ref, idx_ref):
    x = hm_ref[...]
    m = jnp.max(x, axis=1, keepdims=True)
    ii = lax.broadcasted_iota(jnp.int32, x.shape, 1)
    first = jnp.min(jnp.where(x == m, ii, jnp.int32(_HW)), axis=1)
    idx_ref[0, 0, :] = first


def _decode_body(wh_hbm, idx_hbm, imgw_hbm, imgh_hbm, out_hbm,
                 idx_v, w_v, h_v, iw_v, ih_v, out_v, sem):
    nc = 2
    wid = lax.axis_index("s") * nc + lax.axis_index("c")

    @pl.when(wid < _NUM_WORKERS)
    def _():
        base = wid * _FRAMES_PER_WORKER
        pltpu.sync_copy(idx_hbm.at[pl.ds(base, 16)], idx_v)
        pltpu.sync_copy(imgw_hbm.at[pl.ds(base, 16)], iw_v)
        pltpu.sync_copy(imgh_hbm.at[pl.ds(base, 16)], ih_v)
        idx = idx_v[...]
        frame = base + lax.iota(jnp.int32, 16)
        off0 = frame * (2 * _HW) + idx
        off1 = off0 + _HW
        cp0 = pltpu.async_copy(wh_hbm.at[off0], w_v, sem)
        cp1 = pltpu.async_copy(wh_hbm.at[off1], h_v, sem)
        cp0.wait()
        cp1.wait()
        w = w_v[...]
        h = h_v[...]
        xs = (idx % _W).astype(jnp.float32)
        ys = (idx // _W).astype(jnp.float32)
        sw = iw_v[...] * (1.0 / _W)
        sh = ih_v[...] * (1.0 / _H)
        lane4 = lax.iota(jnp.int32, 16) * 4
        plsc.store_scatter(out_v, [lane4], (xs - 0.5 * w) * sw)
        plsc.store_scatter(out_v, [lane4 + 1], (ys - 0.5 * h) * sh)
        plsc.store_scatter(out_v, [lane4 + 2], (xs + 0.5 * w) * sw)
        plsc.store_scatter(out_v, [lane4 + 3], (ys + 0.5 * h) * sh)
        pltpu.sync_copy(out_v, out_hbm.at[pl.ds(base * 4, 64)])


_decode = functools.partial(
    pl.kernel,
    mesh=plsc.VectorSubcoreMesh(core_axis_name="c", subcore_axis_name="s"),
    out_type=jax.ShapeDtypeStruct((_T * 4,), jnp.float32),
    scratch_types=[
        pltpu.VMEM((16,), jnp.int32),
        pltpu.VMEM((16,), jnp.float32),
        pltpu.VMEM((16,), jnp.float32),
        pltpu.VMEM((16,), jnp.float32),
        pltpu.VMEM((16,), jnp.float32),
        pltpu.VMEM((64,), jnp.float32),
        pltpu.SemaphoreType.DMA,
    ],
)(_decode_body)


def kernel(spatial_map, spatial_wh, target_sizes):
    tb = 8
    hm = spatial_map.reshape(_T, _HW)
    idx3 = pl.pallas_call(
        _argmax_body,
        grid=(_T // tb,),
        in_specs=[pl.BlockSpec((tb, _HW), lambda i: (i, 0))],
        out_specs=pl.BlockSpec((1, 1, tb), lambda i: (i, 0, 0)),
        out_shape=jax.ShapeDtypeStruct((_T // tb, 1, tb), jnp.int32),
    )(hm)
    idx = idx3.reshape(_T)

    wh_flat = spatial_wh.reshape(_T * 2 * _HW)
    ts = target_sizes.astype(jnp.float32)
    img_h = ts[:, 0]
    img_w = ts[:, 1]
    boxes_flat = _decode(wh_flat, idx, img_w, img_h)
    return boxes_flat.reshape(_T, 4)


# speedup vs baseline: 1.3952x; 1.3952x over previous
import functools

import jax
import jax.numpy as jnp
from jax import lax
from jax.experimental import pallas as pl
from jax.experimental.pallas import tpu as pltpu
from jax.experimental.pallas import tpu_sc as plsc

_T, _H, _W = 256, 128, 128
_HW = _H * _W
_FRAMES_PER_WORKER = 16
_NUM_WORKERS = _T // _FRAMES_PER_WORKER


def _argmax_body(hm_ref, idx_ref):
    x = hm_ref[...]
    m = jnp.max(x, axis=1, keepdims=True)
    ii = lax.broadcasted_iota(jnp.int32, x.shape, 1)
    first = jnp.min(jnp.where(x == m, ii, jnp.int32(_HW)), axis=1)
    idx_ref[0, 0, :] = first


def _decode_body(wh_hbm, idx_hbm, imgw_hbm, imgh_hbm, out_hbm,
                 idx_v, off0_v, off1_v, w_v, h_v, iw_v, ih_v, out_v, sem):
    nc = 2
    wid = lax.axis_index("s") * nc + lax.axis_index("c")

    @pl.when(wid < _NUM_WORKERS)
    def _():
        base = wid * _FRAMES_PER_WORKER
        pltpu.sync_copy(idx_hbm.at[pl.ds(base, 16)], idx_v)
        pltpu.sync_copy(imgw_hbm.at[pl.ds(base, 16)], iw_v)
        pltpu.sync_copy(imgh_hbm.at[pl.ds(base, 16)], ih_v)
        idx = idx_v[...]
        frame = base + lax.iota(jnp.int32, 16)
        off0 = frame * (2 * _HW) + idx
        off0_v[...] = off0
        off1_v[...] = off0 + _HW
        cp0 = pltpu.async_copy(wh_hbm.at[off0_v], w_v, sem)
        cp1 = pltpu.async_copy(wh_hbm.at[off1_v], h_v, sem)
        cp0.wait()
        cp1.wait()
        w = w_v[...]
        h = h_v[...]
        xs = (idx % _W).astype(jnp.float32)
        ys = (idx >> 7).astype(jnp.float32)
        sw = iw_v[...] * (1.0 / _W)
        sh = ih_v[...] * (1.0 / _H)
        out_v[pl.ds(0, 16)] = (xs - 0.5 * w) * sw
        out_v[pl.ds(16, 16)] = (ys - 0.5 * h) * sh
        out_v[pl.ds(32, 16)] = (xs + 0.5 * w) * sw
        out_v[pl.ds(48, 16)] = (ys + 0.5 * h) * sh
        for c in range(4):
            pltpu.sync_copy(out_v.at[pl.ds(c * 16, 16)],
                            out_hbm.at[pl.ds(c * _T + base, 16)])


@functools.lru_cache(maxsize=None)
def _build_decode():
    return functools.partial(
        pl.kernel,
        mesh=plsc.VectorSubcoreMesh(core_axis_name="c", subcore_axis_name="s"),
        out_type=jax.ShapeDtypeStruct((_T * 4,), jnp.float32),
        scratch_types=[
            pltpu.VMEM((16,), jnp.int32),
            pltpu.VMEM((16,), jnp.int32),
            pltpu.VMEM((16,), jnp.int32),
            pltpu.VMEM((16,), jnp.float32),
            pltpu.VMEM((16,), jnp.float32),
            pltpu.VMEM((16,), jnp.float32),
            pltpu.VMEM((16,), jnp.float32),
            pltpu.VMEM((64,), jnp.float32),
            pltpu.SemaphoreType.DMA,
        ],
    )(_decode_body)


def kernel(spatial_map, spatial_wh, target_sizes):
    tb = 8
    hm = spatial_map.reshape(_T, _HW)
    idx3 = pl.pallas_call(
        _argmax_body,
        grid=(_T // tb,),
        in_specs=[pl.BlockSpec((tb, _HW), lambda i: (i, 0))],
        out_specs=pl.BlockSpec((1, 1, tb), lambda i: (i, 0, 0)),
        out_shape=jax.ShapeDtypeStruct((_T // tb, 1, tb), jnp.int32),
    )(hm)
    idx = idx3.reshape(_T)

    wh_flat = spatial_wh.reshape(_T * 2 * _HW)
    ts = target_sizes.astype(jnp.float32)
    img_h = ts[:, 0]
    img_w = ts[:, 1]
    boxes_cm = _build_decode()(wh_flat, idx, img_w, img_h)
    return boxes_cm.reshape(4, _T).T


# speedup vs baseline: 1.6186x; 1.1601x over previous
import functools

import jax
import jax.numpy as jnp
from jax import lax
from jax.experimental import pallas as pl
from jax.experimental.pallas import tpu as pltpu
from jax.experimental.pallas import tpu_sc as plsc

_T, _H, _W = 256, 128, 128
_HW = _H * _W
_FRAMES_PER_WORKER = 16
_NUM_WORKERS = _T // _FRAMES_PER_WORKER


def _argmax_body(hm_ref, idx_ref):
    x = hm_ref[:, 0]
    m = jnp.max(jnp.max(x, axis=2), axis=1)
    iy = lax.broadcasted_iota(jnp.int32, x.shape, 1)
    ix = lax.broadcasted_iota(jnp.int32, x.shape, 2)
    flat = iy * _W + ix
    hit = jnp.where(x == m[:, None, None], flat, jnp.int32(_HW))
    first = jnp.min(jnp.min(hit, axis=2), axis=1)
    idx_ref[0, 0, :] = first


def _decode_body(wh_hbm, idx_hbm, imgw_hbm, imgh_hbm, out_hbm,
                 idx_v, off0_v, off1_v, w_v, h_v, iw_v, ih_v, out_v, sem):
    nc = 2
    wid = lax.axis_index("s") * nc + lax.axis_index("c")

    @pl.when(wid < _NUM_WORKERS)
    def _():
        base = wid * _FRAMES_PER_WORKER
        pltpu.sync_copy(idx_hbm.at[pl.ds(base, 16)], idx_v)
        pltpu.sync_copy(imgw_hbm.at[pl.ds(base, 16)], iw_v)
        pltpu.sync_copy(imgh_hbm.at[pl.ds(base, 16)], ih_v)
        idx = idx_v[...]
        frame = base + lax.iota(jnp.int32, 16)
        off0 = frame * (2 * _HW) + idx
        off0_v[...] = off0
        off1_v[...] = off0 + _HW
        cp0 = pltpu.async_copy(wh_hbm.at[off0_v], w_v, sem)
        cp1 = pltpu.async_copy(wh_hbm.at[off1_v], h_v, sem)
        cp0.wait()
        cp1.wait()
        w = w_v[...]
        h = h_v[...]
        xs = (idx % _W).astype(jnp.float32)
        ys = (idx >> 7).astype(jnp.float32)
        sw = iw_v[...] * (1.0 / _W)
        sh = ih_v[...] * (1.0 / _H)
        out_v[pl.ds(0, 16)] = (xs - 0.5 * w) * sw
        out_v[pl.ds(16, 16)] = (ys - 0.5 * h) * sh
        out_v[pl.ds(32, 16)] = (xs + 0.5 * w) * sw
        out_v[pl.ds(48, 16)] = (ys + 0.5 * h) * sh
        for c in range(4):
            pltpu.sync_copy(out_v.at[pl.ds(c * 16, 16)],
                            out_hbm.at[pl.ds(c * _T + base, 16)])


@functools.lru_cache(maxsize=None)
def _build_decode():
    return functools.partial(
        pl.kernel,
        mesh=plsc.VectorSubcoreMesh(core_axis_name="c", subcore_axis_name="s"),
        out_type=jax.ShapeDtypeStruct((_T * 4,), jnp.float32),
        scratch_types=[
            pltpu.VMEM((16,), jnp.int32),
            pltpu.VMEM((16,), jnp.int32),
            pltpu.VMEM((16,), jnp.int32),
            pltpu.VMEM((16,), jnp.float32),
            pltpu.VMEM((16,), jnp.float32),
            pltpu.VMEM((16,), jnp.float32),
            pltpu.VMEM((16,), jnp.float32),
            pltpu.VMEM((64,), jnp.float32),
            pltpu.SemaphoreType.DMA,
        ],
    )(_decode_body)


def kernel(spatial_map, spatial_wh, target_sizes):
    tb = 8
    idx3 = pl.pallas_call(
        _argmax_body,
        grid=(_T // tb,),
        in_specs=[pl.BlockSpec((tb, 1, _H, _W), lambda i: (i, 0, 0, 0))],
        out_specs=pl.BlockSpec((1, 1, tb), lambda i: (i, 0, 0)),
        out_shape=jax.ShapeDtypeStruct((_T // tb, 1, tb), jnp.int32),
    )(spatial_map)
    idx = idx3.reshape(_T)

    wh_flat = spatial_wh.reshape(_T * 2 * _HW)
    ts = target_sizes.astype(jnp.float32)
    img_h = ts[:, 0]
    img_w = ts[:, 1]
    boxes_cm = _build_decode()(wh_flat, idx, img_w, img_h)
    return boxes_cm.reshape(4, _T).T


# speedup vs baseline: 1.8983x; 1.1728x over previous
import functools

import jax
import jax.numpy as jnp
from jax import lax
from jax.experimental import pallas as pl
from jax.experimental.pallas import tpu as pltpu
from jax.experimental.pallas import tpu_sc as plsc

_T, _H, _W = 256, 128, 128
_HW = _H * _W
_FRAMES_PER_WORKER = 16
_NUM_WORKERS = _T // _FRAMES_PER_WORKER


def _argmax_body(hm_ref, idx_ref):
    x = hm_ref[:, 0]
    m = jnp.max(jnp.max(x, axis=1), axis=1)
    iy = lax.broadcasted_iota(jnp.int32, x.shape, 1)
    ix = lax.broadcasted_iota(jnp.int32, x.shape, 2)
    flat = iy * _W + ix
    hit = jnp.where(x == m[:, None, None], flat, jnp.int32(_HW))
    first = jnp.min(jnp.min(hit, axis=1), axis=1)
    idx_ref[0, 0, :] = first


def _decode_body(wh_hbm, idx_hbm, imgw_hbm, imgh_hbm, out_hbm,
                 idx_v, off0_v, off1_v, w_v, h_v, iw_v, ih_v, out_v, sem):
    nc = 2
    wid = lax.axis_index("s") * nc + lax.axis_index("c")

    @pl.when(wid < _NUM_WORKERS)
    def _():
        base = wid * _FRAMES_PER_WORKER
        pltpu.sync_copy(idx_hbm.at[pl.ds(base, 16)], idx_v)
        pltpu.sync_copy(imgw_hbm.at[pl.ds(base, 16)], iw_v)
        pltpu.sync_copy(imgh_hbm.at[pl.ds(base, 16)], ih_v)
        idx = idx_v[...]
        frame = base + lax.iota(jnp.int32, 16)
        off0 = frame * (2 * _HW) + idx
        off0_v[...] = off0
        off1_v[...] = off0 + _HW
        cp0 = pltpu.async_copy(wh_hbm.at[off0_v], w_v, sem)
        cp1 = pltpu.async_copy(wh_hbm.at[off1_v], h_v, sem)
        cp0.wait()
        cp1.wait()
        w = w_v[...]
        h = h_v[...]
        xs = (idx % _W).astype(jnp.float32)
        ys = (idx >> 7).astype(jnp.float32)
        sw = iw_v[...] * (1.0 / _W)
        sh = ih_v[...] * (1.0 / _H)
        out_v[pl.ds(0, 16)] = (xs - 0.5 * w) * sw
        out_v[pl.ds(16, 16)] = (ys - 0.5 * h) * sh
        out_v[pl.ds(32, 16)] = (xs + 0.5 * w) * sw
        out_v[pl.ds(48, 16)] = (ys + 0.5 * h) * sh
        for c in range(4):
            pltpu.sync_copy(out_v.at[pl.ds(c * 16, 16)],
                            out_hbm.at[pl.ds(c * _T + base, 16)])


@functools.lru_cache(maxsize=None)
def _build_decode():
    return functools.partial(
        pl.kernel,
        mesh=plsc.VectorSubcoreMesh(core_axis_name="c", subcore_axis_name="s"),
        out_type=jax.ShapeDtypeStruct((_T * 4,), jnp.float32),
        scratch_types=[
            pltpu.VMEM((16,), jnp.int32),
            pltpu.VMEM((16,), jnp.int32),
            pltpu.VMEM((16,), jnp.int32),
            pltpu.VMEM((16,), jnp.float32),
            pltpu.VMEM((16,), jnp.float32),
            pltpu.VMEM((16,), jnp.float32),
            pltpu.VMEM((16,), jnp.float32),
            pltpu.VMEM((64,), jnp.float32),
            pltpu.SemaphoreType.DMA,
        ],
    )(_decode_body)


def kernel(spatial_map, spatial_wh, target_sizes):
    tb = 8
    idx3 = pl.pallas_call(
        _argmax_body,
        grid=(_T // tb,),
        in_specs=[pl.BlockSpec((tb, 1, _H, _W), lambda i: (i, 0, 0, 0))],
        out_specs=pl.BlockSpec((1, 1, tb), lambda i: (i, 0, 0)),
        out_shape=jax.ShapeDtypeStruct((_T // tb, 1, tb), jnp.int32),
    )(spatial_map)
    idx = idx3.reshape(_T)

    wh_flat = spatial_wh.reshape(_T * 2 * _HW)
    ts = target_sizes.astype(jnp.float32)
    img_h = ts[:, 0]
    img_w = ts[:, 1]
    boxes_cm = _build_decode()(wh_flat, idx, img_w, img_h)
    return boxes_cm.reshape(4, _T).T


# speedup vs baseline: 2.6690x; 1.4060x over previous
import functools

import jax
import jax.numpy as jnp
from jax import lax
from jax.experimental import pallas as pl
from jax.experimental.pallas import tpu as pltpu
from jax.experimental.pallas import tpu_sc as plsc

_T, _H, _W = 256, 128, 128
_HW = _H * _W
_FRAMES_PER_WORKER = 16
_NUM_WORKERS = _T // _FRAMES_PER_WORKER


def _argmax_body(hm_ref, idx_ref):
    x = hm_ref[:, 0]
    m = jnp.max(jnp.max(x, axis=1), axis=1)
    iy = lax.broadcasted_iota(jnp.int32, x.shape, 1)
    ix = lax.broadcasted_iota(jnp.int32, x.shape, 2)
    flat = iy * _W + ix
    hit = jnp.where(x == m[:, None, None], flat, jnp.int32(_HW))
    first = jnp.min(jnp.min(hit, axis=1), axis=1)
    idx_ref[0, 0, :] = first


def _decode_body(wh_hbm, idx_hbm, imgw_hbm, imgh_hbm, out_hbm,
                 idx_v, off0_v, off1_v, w_v, h_v, iw_v, ih_v, out_v, sem):
    nc = 2
    wid = lax.axis_index("s") * nc + lax.axis_index("c")

    @pl.when(wid < _NUM_WORKERS)
    def _():
        base = wid * _FRAMES_PER_WORKER
        pltpu.sync_copy(idx_hbm.at[pl.ds(base, 16)], idx_v)
        pltpu.sync_copy(imgw_hbm.at[pl.ds(base, 16)], iw_v)
        pltpu.sync_copy(imgh_hbm.at[pl.ds(base, 16)], ih_v)
        idx = idx_v[...]
        frame = base + lax.iota(jnp.int32, 16)
        off0 = frame * (2 * _HW) + idx
        off0_v[...] = off0
        off1_v[...] = off0 + _HW
        cp0 = pltpu.async_copy(wh_hbm.at[off0_v], w_v, sem)
        cp1 = pltpu.async_copy(wh_hbm.at[off1_v], h_v, sem)
        cp0.wait()
        cp1.wait()
        w = w_v[...]
        h = h_v[...]
        xs = (idx % _W).astype(jnp.float32)
        ys = (idx >> 7).astype(jnp.float32)
        sw = iw_v[...] * (1.0 / _W)
        sh = ih_v[...] * (1.0 / _H)
        out_v[pl.ds(0, 16)] = (xs - 0.5 * w) * sw
        out_v[pl.ds(16, 16)] = (ys - 0.5 * h) * sh
        out_v[pl.ds(32, 16)] = (xs + 0.5 * w) * sw
        out_v[pl.ds(48, 16)] = (ys + 0.5 * h) * sh
        for c in range(4):
            pltpu.sync_copy(out_v.at[pl.ds(c * 16, 16)],
                            out_hbm.at[pl.ds(c * _T + base, 16)])


@functools.lru_cache(maxsize=None)
def _build_decode():
    return functools.partial(
        pl.kernel,
        mesh=plsc.VectorSubcoreMesh(core_axis_name="c", subcore_axis_name="s"),
        out_type=jax.ShapeDtypeStruct((_T * 4,), jnp.float32),
        scratch_types=[
            pltpu.VMEM((16,), jnp.int32),
            pltpu.VMEM((16,), jnp.int32),
            pltpu.VMEM((16,), jnp.int32),
            pltpu.VMEM((16,), jnp.float32),
            pltpu.VMEM((16,), jnp.float32),
            pltpu.VMEM((16,), jnp.float32),
            pltpu.VMEM((16,), jnp.float32),
            pltpu.VMEM((64,), jnp.float32),
            pltpu.SemaphoreType.DMA,
        ],
    )(_decode_body)


def kernel(spatial_map, spatial_wh, target_sizes):
    tb = 32
    idx3 = pl.pallas_call(
        _argmax_body,
        grid=(_T // tb,),
        in_specs=[pl.BlockSpec((tb, 1, _H, _W), lambda i: (i, 0, 0, 0))],
        out_specs=pl.BlockSpec((1, 1, tb), lambda i: (i, 0, 0)),
        out_shape=jax.ShapeDtypeStruct((_T // tb, 1, tb), jnp.int32),
    )(spatial_map)
    idx = idx3.reshape(_T)

    wh_flat = spatial_wh.reshape(_T * 2 * _HW)
    ts = target_sizes.astype(jnp.float32)
    img_h = ts[:, 0]
    img_w = ts[:, 1]
    boxes_cm = _build_decode()(wh_flat, idx, img_w, img_h)
    return boxes_cm.reshape(4, _T).T


# speedup vs baseline: 2.8641x; 1.0731x over previous
import functools

import jax
import jax.numpy as jnp
from jax import lax
from jax.experimental import pallas as pl
from jax.experimental.pallas import tpu as pltpu
from jax.experimental.pallas import tpu_sc as plsc

_T, _H, _W = 256, 128, 128
_HW = _H * _W
_FRAMES_PER_WORKER = 16
_NUM_WORKERS = _T // _FRAMES_PER_WORKER


def _argmax_body(hm_ref, idx_ref):
    x = hm_ref[:, 0]
    m = jnp.max(jnp.max(x, axis=1), axis=1)
    iy = lax.broadcasted_iota(jnp.int32, x.shape, 1)
    ix = lax.broadcasted_iota(jnp.int32, x.shape, 2)
    flat = iy * _W + ix
    hit = jnp.where(x == m[:, None, None], flat, jnp.int32(_HW))
    first = jnp.min(jnp.min(hit, axis=1), axis=1)
    idx_ref[0, 0, :] = first


def _decode_body(wh_hbm, idx_hbm, imgw_hbm, imgh_hbm, out_hbm,
                 idx_v, off0_v, off1_v, w_v, h_v, iw_v, ih_v, out_v, sem):
    nc = 2
    wid = lax.axis_index("s") * nc + lax.axis_index("c")

    @pl.when(wid < _NUM_WORKERS)
    def _():
        base = wid * _FRAMES_PER_WORKER
        pltpu.sync_copy(idx_hbm.at[pl.ds(base, 16)], idx_v)
        pltpu.sync_copy(imgw_hbm.at[pl.ds(base, 16)], iw_v)
        pltpu.sync_copy(imgh_hbm.at[pl.ds(base, 16)], ih_v)
        idx = idx_v[...]
        frame = base + lax.iota(jnp.int32, 16)
        off0 = frame * (2 * _HW) + idx
        off0_v[...] = off0
        off1_v[...] = off0 + _HW
        cp0 = pltpu.async_copy(wh_hbm.at[off0_v], w_v, sem)
        cp1 = pltpu.async_copy(wh_hbm.at[off1_v], h_v, sem)
        cp0.wait()
        cp1.wait()
        w = w_v[...]
        h = h_v[...]
        xs = (idx % _W).astype(jnp.float32)
        ys = (idx >> 7).astype(jnp.float32)
        sw = iw_v[...] * (1.0 / _W)
        sh = ih_v[...] * (1.0 / _H)
        out_v[pl.ds(0, 16)] = (xs - 0.5 * w) * sw
        out_v[pl.ds(16, 16)] = (ys - 0.5 * h) * sh
        out_v[pl.ds(32, 16)] = (xs + 0.5 * w) * sw
        out_v[pl.ds(48, 16)] = (ys + 0.5 * h) * sh
        for c in range(4):
            pltpu.sync_copy(out_v.at[pl.ds(c * 16, 16)],
                            out_hbm.at[pl.ds(c * _T + base, 16)])


@functools.lru_cache(maxsize=None)
def _build_decode():
    return functools.partial(
        pl.kernel,
        mesh=plsc.VectorSubcoreMesh(core_axis_name="c", subcore_axis_name="s"),
        out_type=jax.ShapeDtypeStruct((_T * 4,), jnp.float32),
        scratch_types=[
            pltpu.VMEM((16,), jnp.int32),
            pltpu.VMEM((16,), jnp.int32),
            pltpu.VMEM((16,), jnp.int32),
            pltpu.VMEM((16,), jnp.float32),
            pltpu.VMEM((16,), jnp.float32),
            pltpu.VMEM((16,), jnp.float32),
            pltpu.VMEM((16,), jnp.float32),
            pltpu.VMEM((64,), jnp.float32),
            pltpu.SemaphoreType.DMA,
        ],
    )(_decode_body)


def kernel(spatial_map, spatial_wh, target_sizes):
    tb = 64
    idx3 = pl.pallas_call(
        _argmax_body,
        grid=(_T // tb,),
        in_specs=[pl.BlockSpec((tb, 1, _H, _W), lambda i: (i, 0, 0, 0))],
        out_specs=pl.BlockSpec((1, 1, tb), lambda i: (i, 0, 0)),
        out_shape=jax.ShapeDtypeStruct((_T // tb, 1, tb), jnp.int32),
    )(spatial_map)
    idx = idx3.reshape(_T)

    wh_flat = spatial_wh.reshape(_T * 2 * _HW)
    ts = target_sizes.astype(jnp.float32)
    img_h = ts[:, 0]
    img_w = ts[:, 1]
    boxes_cm = _build_decode()(wh_flat, idx, img_w, img_h)
    return boxes_cm.reshape(4, _T).T


# speedup vs baseline: 2.9458x; 1.0285x over previous
import functools

import jax
import jax.numpy as jnp
from jax import lax
from jax.experimental import pallas as pl
from jax.experimental.pallas import tpu as pltpu
from jax.experimental.pallas import tpu_sc as plsc

_T, _H, _W = 256, 128, 128
_HW = _H * _W
_FRAMES_PER_WORKER = 16
_NUM_WORKERS = _T // _FRAMES_PER_WORKER


def _argmax_body(hm_ref, idx_ref):
    x = hm_ref[:, 0]
    m = jnp.max(jnp.max(x, axis=1), axis=1)
    iy = lax.broadcasted_iota(jnp.int32, x.shape, 1)
    ix = lax.broadcasted_iota(jnp.int32, x.shape, 2)
    flat = iy * _W + ix
    hit = jnp.where(x == m[:, None, None], flat, jnp.int32(_HW))
    first = jnp.min(jnp.min(hit, axis=1), axis=1)
    idx_ref[0, 0, :] = first


def _decode_body(wh_hbm, idx_hbm, imgw_hbm, imgh_hbm, out_hbm,
                 idx_v, off0_v, off1_v, w_v, h_v, iw_v, ih_v, out_v, sem):
    nc = 2
    wid = lax.axis_index("s") * nc + lax.axis_index("c")

    @pl.when(wid < _NUM_WORKERS)
    def _():
        base = wid * _FRAMES_PER_WORKER
        cpi = pltpu.async_copy(idx_hbm.at[pl.ds(base, 16)], idx_v, sem)
        cpw = pltpu.async_copy(imgw_hbm.at[pl.ds(base, 16)], iw_v, sem)
        cph = pltpu.async_copy(imgh_hbm.at[pl.ds(base, 16)], ih_v, sem)
        cpi.wait()
        cpw.wait()
        cph.wait()
        idx = idx_v[...]
        frame = base + lax.iota(jnp.int32, 16)
        off0 = frame * (2 * _HW) + idx
        off0_v[...] = off0
        off1_v[...] = off0 + _HW
        cp0 = pltpu.async_copy(wh_hbm.at[off0_v], w_v, sem)
        cp1 = pltpu.async_copy(wh_hbm.at[off1_v], h_v, sem)
        cp0.wait()
        cp1.wait()
        w = w_v[...]
        h = h_v[...]
        xs = (idx % _W).astype(jnp.float32)
        ys = (idx >> 7).astype(jnp.float32)
        sw = iw_v[...] * (1.0 / _W)
        sh = ih_v[...] * (1.0 / _H)
        out_v[pl.ds(0, 16)] = (xs - 0.5 * w) * sw
        out_v[pl.ds(16, 16)] = (ys - 0.5 * h) * sh
        out_v[pl.ds(32, 16)] = (xs + 0.5 * w) * sw
        out_v[pl.ds(48, 16)] = (ys + 0.5 * h) * sh
        outs = [pltpu.async_copy(out_v.at[pl.ds(c * 16, 16)],
                                 out_hbm.at[pl.ds(c * _T + base, 16)], sem)
                for c in range(4)]
        for cp in outs:
            cp.wait()


@functools.lru_cache(maxsize=None)
def _build_decode():
    return functools.partial(
        pl.kernel,
        mesh=plsc.VectorSubcoreMesh(core_axis_name="c", subcore_axis_name="s"),
        out_type=jax.ShapeDtypeStruct((_T * 4,), jnp.float32),
        scratch_types=[
            pltpu.VMEM((16,), jnp.int32),
            pltpu.VMEM((16,), jnp.int32),
            pltpu.VMEM((16,), jnp.int32),
            pltpu.VMEM((16,), jnp.float32),
            pltpu.VMEM((16,), jnp.float32),
            pltpu.VMEM((16,), jnp.float32),
            pltpu.VMEM((16,), jnp.float32),
            pltpu.VMEM((64,), jnp.float32),
            pltpu.SemaphoreType.DMA,
        ],
    )(_decode_body)


def kernel(spatial_map, spatial_wh, target_sizes):
    tb = 64
    idx3 = pl.pallas_call(
        _argmax_body,
        grid=(_T // tb,),
        in_specs=[pl.BlockSpec((tb, 1, _H, _W), lambda i: (i, 0, 0, 0))],
        out_specs=pl.BlockSpec((1, 1, tb), lambda i: (i, 0, 0)),
        out_shape=jax.ShapeDtypeStruct((_T // tb, 1, tb), jnp.int32),
    )(spatial_map)
    idx = idx3.reshape(_T)

    wh_flat = spatial_wh.reshape(_T * 2 * _HW)
    ts = target_sizes.astype(jnp.float32)
    img_h = ts[:, 0]
    img_w = ts[:, 1]
    boxes_cm = _build_decode()(wh_flat, idx, img_w, img_h)
    return boxes_cm.reshape(4, _T).T


# speedup vs baseline: 3.2282x; 1.0959x over previous
import functools

import jax
import jax.numpy as jnp
from jax import lax
from jax.experimental import pallas as pl
from jax.experimental.pallas import tpu as pltpu
from jax.experimental.pallas import tpu_sc as plsc

_T, _H, _W = 256, 128, 128
_HW = _H * _W
_FRAMES_PER_WORKER = 16
_NUM_WORKERS = _T // _FRAMES_PER_WORKER


def _argmax_body(hm_ref, idx_ref):
    x = hm_ref[:, 0]
    m = jnp.max(jnp.max(x, axis=1), axis=1)
    iy = lax.broadcasted_iota(jnp.int32, x.shape, 1)
    ix = lax.broadcasted_iota(jnp.int32, x.shape, 2)
    flat = iy * _W + ix
    hit = jnp.where(x == m[:, None, None], flat, jnp.int32(_HW))
    first = jnp.min(jnp.min(hit, axis=1), axis=1)
    idx_ref[0, 0, :] = first


def _decode_body(wh_hbm, idx_hbm, imgw_hbm, imgh_hbm, out_hbm,
                 idx_v, off0_v, off1_v, w_v, h_v, iw_v, ih_v, out_v, sem):
    nc = 1
    wid = lax.axis_index("s") * nc + lax.axis_index("c")

    @pl.when(wid < _NUM_WORKERS)
    def _():
        base = wid * _FRAMES_PER_WORKER
        cpi = pltpu.async_copy(idx_hbm.at[pl.ds(base, 16)], idx_v, sem)
        cpw = pltpu.async_copy(imgw_hbm.at[pl.ds(base, 16)], iw_v, sem)
        cph = pltpu.async_copy(imgh_hbm.at[pl.ds(base, 16)], ih_v, sem)
        cpi.wait()
        cpw.wait()
        cph.wait()
        idx = idx_v[...]
        frame = base + lax.iota(jnp.int32, 16)
        off0 = frame * (2 * _HW) + idx
        off0_v[...] = off0
        off1_v[...] = off0 + _HW
        cp0 = pltpu.async_copy(wh_hbm.at[off0_v], w_v, sem)
        cp1 = pltpu.async_copy(wh_hbm.at[off1_v], h_v, sem)
        cp0.wait()
        cp1.wait()
        w = w_v[...]
        h = h_v[...]
        xs = (idx % _W).astype(jnp.float32)
        ys = (idx >> 7).astype(jnp.float32)
        sw = iw_v[...] * (1.0 / _W)
        sh = ih_v[...] * (1.0 / _H)
        out_v[pl.ds(0, 16)] = (xs - 0.5 * w) * sw
        out_v[pl.ds(16, 16)] = (ys - 0.5 * h) * sh
        out_v[pl.ds(32, 16)] = (xs + 0.5 * w) * sw
        out_v[pl.ds(48, 16)] = (ys + 0.5 * h) * sh
        obase = pl.multiple_of((base >> 7) * 512 + (base & 127), 16)
        outs = [pltpu.async_copy(out_v.at[pl.ds(c * 16, 16)],
                                 out_hbm.at[pl.ds(obase + c * 128, 16)], sem)
                for c in range(4)]
        for cp in outs:
            cp.wait()


@functools.lru_cache(maxsize=None)
def _build_decode():
    return functools.partial(
        pl.kernel,
        mesh=plsc.VectorSubcoreMesh(core_axis_name="c", subcore_axis_name="s",
                                    num_cores=1),
        out_type=jax.ShapeDtypeStruct((_T * 4,), jnp.float32),
        scratch_types=[
            pltpu.VMEM((16,), jnp.int32),
            pltpu.VMEM((16,), jnp.int32),
            pltpu.VMEM((16,), jnp.int32),
            pltpu.VMEM((16,), jnp.float32),
            pltpu.VMEM((16,), jnp.float32),
            pltpu.VMEM((16,), jnp.float32),
            pltpu.VMEM((16,), jnp.float32),
            pltpu.VMEM((64,), jnp.float32),
            pltpu.SemaphoreType.DMA,
        ],
    )(_decode_body)


def kernel(spatial_map, spatial_wh, target_sizes):
    tb = 64
    idx3 = pl.pallas_call(
        _argmax_body,
        grid=(_T // tb,),
        in_specs=[pl.BlockSpec((tb, 1, _H, _W), lambda i: (i, 0, 0, 0))],
        out_specs=pl.BlockSpec((1, 1, tb), lambda i: (i, 0, 0)),
        out_shape=jax.ShapeDtypeStruct((_T // tb, 1, tb), jnp.int32),
    )(spatial_map)
    idx = idx3.reshape(_T)

    wh_flat = spatial_wh.reshape(_T * 2 * _HW)
    ts = target_sizes.astype(jnp.float32)
    img_h = ts[:, 0]
    img_w = ts[:, 1]
    boxes_flat = _build_decode()(wh_flat, idx, img_w, img_h)
    return boxes_flat.reshape(2, 4, 128).transpose(0, 2, 1).reshape(_T, 4)
